# Initial kernel scaffold; baseline (speedup 1.0000x reference)
import jax
import jax.numpy as jnp
from jax.experimental import pallas as pl


def kernel(x, coords, W1, W2, W3):
    raise NotImplementedError("write your pallas kernel here")


# speedup vs baseline: 5.2504x; 5.2504x over previous
import functools

import jax
import jax.numpy as jnp
from jax import lax
from jax.experimental import pallas as pl
from jax.experimental.pallas import tpu as pltpu
from jax.experimental.pallas import tpu_sc as plsc

G = 128
G3 = G * G * G
C3 = 64 * 64 * 64
OFFS = [(dx, dy, dz) for dx in (-1, 0, 1) for dy in (-1, 0, 1) for dz in (-1, 0, 1)]

NC, NS = 2, 16
NW = NC * NS

TBL_LEN = G3 + 32
OCC_LEN = C3 + 32
T_DUMP_LOOKUP = G3 + 8
B_DUMP = C3

N_PTS = 100000
NP_PAD = 100352
PPT_A = NP_PAD // NS
PPT_C = NP_PAD // NW
P_CH = 448
N_CH = PPT_C // P_CH
LPC = 27 * P_CH

BD = 2048
N_BLK = NP_PAD // BD

_i32 = jnp.int32
_f32 = jnp.float32


def _iota16():
    return lax.broadcasted_iota(_i32, (16,), 0)


def _sc_tables_body(cx_h, cy_h, cz_h, tbl_o, occ_o, buf, cxv, cyv, czv,
                    tl_v, ol_v, val_v, sem):
    c = lax.axis_index("c")
    s = lax.axis_index("s")
    tid = c * NS + s

    def fill_buf(i, val):
        def body(g, _):
            buf[pl.ds(g * 16, 16)] = jnp.full((16,), val, _i32)
            return 0
        lax.fori_loop(0, i, body, 0)

    fill_buf(1024, -1)
    tbase = tid * (G3 // NW)
    for i in range(4):
        pltpu.sync_copy(buf, tbl_o.at[pl.ds(tbase + i * 16384, 16384)])

    @pl.when(s == 0)
    def _():
        pltpu.sync_copy(buf.at[pl.ds(0, 16)], tbl_o.at[pl.ds(G3 + 16 * c, 16)])

    fill_buf(1024, 0)
    obase = tid * (C3 // NW)
    pltpu.sync_copy(buf.at[pl.ds(0, 8192)], occ_o.at[pl.ds(obase, 8192)])

    @pl.when(s == 0)
    def _():
        pltpu.sync_copy(buf.at[pl.ds(0, 16)], occ_o.at[pl.ds(C3 + 16 * c, 16)])

    plsc.subcore_barrier()

    pbase = s * PPT_A
    pltpu.sync_copy(cx_h.at[pl.ds(pbase, PPT_A)], cxv)
    pltpu.sync_copy(cy_h.at[pl.ds(pbase, PPT_A)], cyv)
    pltpu.sync_copy(cz_h.at[pl.ds(pbase, PPT_A)], czv)

    t_lo = c * (G3 // 2)
    o_lo = c * (C3 // 2)
    t_dump = G3 + 16 * c
    o_dump = C3 + 16 * c

    def idx_body(g, _):
        px = cxv[pl.ds(g * 16, 16)]
        py = cyv[pl.ds(g * 16, 16)]
        pz = czv[pl.ds(g * 16, 16)]
        lin = (px * G + py) * G + pz
        own = (lin >= t_lo) & (lin < t_lo + G3 // 2)
        tl_v[pl.ds(g * 16, 16)] = jnp.where(own, lin, t_dump)
        cell = ((px >> 1) * 64 + (py >> 1)) * 64 + (pz >> 1)
        owno = (cell >= o_lo) & (cell < o_lo + C3 // 2)
        ol_v[pl.ds(g * 16, 16)] = jnp.where(owno, cell, o_dump)
        val_v[pl.ds(g * 16, 16)] = pbase + g * 16 + _iota16()
        return 0

    lax.fori_loop(0, PPT_A // 16, idx_body, 0)
    pltpu.async_copy(val_v, tbl_o.at[tl_v], sem).wait()

    def ones_body(g, _):
        val_v[pl.ds(g * 16, 16)] = jnp.full((16,), 1, _i32)
        return 0

    lax.fori_loop(0, PPT_A // 16, ones_body, 0)
    pltpu.async_copy(val_v, occ_o.at[ol_v], sem).wait()


def _sc_tables(cx, cy, cz):
    mesh = plsc.VectorSubcoreMesh(core_axis_name="c", subcore_axis_name="s")
    fn = pl.kernel(
        _sc_tables_body,
        out_type=[jax.ShapeDtypeStruct((TBL_LEN,), _i32),
                  jax.ShapeDtypeStruct((OCC_LEN,), _i32)],
        mesh=mesh,
        scratch_types=[pltpu.VMEM((16384,), _i32),
                       pltpu.VMEM((PPT_A,), _i32),
                       pltpu.VMEM((PPT_A,), _i32),
                       pltpu.VMEM((PPT_A,), _i32),
                       pltpu.VMEM((PPT_A,), _i32),
                       pltpu.VMEM((PPT_A,), _i32),
                       pltpu.VMEM((PPT_A,), _i32),
                       pltpu.SemaphoreType.DMA],
    )
    return fn(cx, cy, cz)


def _tc_bmask_body(occ_ref, b_ref, nlive_ref):
    occ = occ_ref[...]

    def sh(a, axis, d):
        if d == 0:
            return a
        z_shape = list(a.shape)
        z_shape[axis] = 1
        z = jnp.zeros(z_shape, _i32)
        if d == 1:
            sl = [slice(None)] * 3
            sl[axis] = slice(0, a.shape[axis] - 1)
            return jnp.concatenate([z, a[tuple(sl)]], axis=axis)
        sl = [slice(None)] * 3
        sl[axis] = slice(1, None)
        return jnp.concatenate([a[tuple(sl)], z], axis=axis)

    acc = jnp.zeros((64, 64, 64), _i32)
    xs = {d: sh(occ, 0, d) for d in (-1, 0, 1)}
    for dx in (-1, 0, 1):
        ys = {d: sh(xs[dx], 1, d) for d in (-1, 0, 1)}
        for dy in (-1, 0, 1):
            for dz in (-1, 0, 1):
                k = ((dx + 1) * 3 + (dy + 1)) * 3 + (dz + 1)
                acc = acc + (sh(ys[dy], 2, dz) << k)
    b_ref[...] = acc * occ
    nlive_ref[0, 0] = jnp.sum(occ)


def _tc_bmask(occ3):
    return pl.pallas_call(
        _tc_bmask_body,
        grid=(),
        in_specs=[pl.BlockSpec((64, 64, 64), lambda: (0, 0, 0))],
        out_specs=[pl.BlockSpec((64, 64, 64), lambda: (0, 0, 0)),
                   pl.BlockSpec(memory_space=pltpu.SMEM)],
        out_shape=[jax.ShapeDtypeStruct((64, 64, 64), _i32),
                   jax.ShapeDtypeStruct((1, 1), _i32)],
    )(occ3)


def _sc_gather_body(cx_h, cy_h, cz_h, tbl_h, bpad_h, x0_h, x1_h, x2_h, x3_h,
                    gxp_o, bj_o, cxv, cyv, czv, ilist, vals, pbuf, sem):
    c = lax.axis_index("c")
    s = lax.axis_index("s")
    tid = c * NS + s
    base = tid * PPT_C

    def chunk(ci, _):
        cbase = base + ci * P_CH
        pltpu.sync_copy(cx_h.at[pl.ds(cbase, P_CH)], cxv)
        pltpu.sync_copy(cy_h.at[pl.ds(cbase, P_CH)], cyv)
        pltpu.sync_copy(cz_h.at[pl.ds(cbase, P_CH)], czv)

        def g1(g, _):
            px = cxv[pl.ds(g * 16, 16)]
            py = cyv[pl.ds(g * 16, 16)]
            pz = czv[pl.ds(g * 16, 16)]
            for i, (dx, dy, dz) in enumerate(OFFS):
                nx, ny, nz = px + dx, py + dy, pz + dz
                inb = ((nx >= 0) & (nx < G) & (ny >= 0) & (ny < G)
                       & (nz >= 0) & (nz < G))
                lin = (nx * G + ny) * G + nz
                addr = jnp.where(inb, lin, T_DUMP_LOOKUP)
                ilist[pl.ds(i * P_CH + g * 16, 16)] = addr
            return 0

        lax.fori_loop(0, P_CH // 16, g1, 0)
        pltpu.async_copy(tbl_h.at[ilist], vals, sem).wait()

        def fix(t, _):
            v = vals[pl.ds(t * 16, 16)]
            ilist[pl.ds(t * 16, 16)] = jnp.where(v < 0, N_PTS, v)
            return 0

        lax.fori_loop(0, LPC // 16, fix, 0)
        for d, xd_h in enumerate((x0_h, x1_h, x2_h, x3_h)):
            pltpu.async_copy(xd_h.at[ilist], pbuf, sem).wait()
            pltpu.sync_copy(pbuf, gxp_o.at[d, pl.ds(cbase * 27, LPC)])

        def g2(g, _):
            px = cxv[pl.ds(g * 16, 16)]
            py = cyv[pl.ds(g * 16, 16)]
            pz = czv[pl.ds(g * 16, 16)]
            for j, (dx, dy, dz) in enumerate(OFFS):
                vx, vy, vz = px - dx, py - dy, pz - dz
                ok = (((vx | vy | vz) & 1) == 0)
                ok = ok & (vx >= 0) & (vy >= 0) & (vz >= 0)
                ok = ok & (vx < G) & (vy < G) & (vz < G)
                cell = ((vx >> 1) * 64 + (vy >> 1)) * 64 + (vz >> 1)
                ilist[pl.ds(j * P_CH + g * 16, 16)] = jnp.where(ok, cell,
                                                                B_DUMP)
            return 0

        lax.fori_loop(0, P_CH // 16, g2, 0)
        pltpu.async_copy(bpad_h.at[ilist], vals, sem).wait()
        pltpu.sync_copy(vals, bj_o.at[pl.ds(cbase * 27, LPC)])
        return 0

    lax.fori_loop(0, N_CH, chunk, 0)


def _sc_gather(cx, cy, cz, tbl, bpad, x0, x1, x2, x3):
    mesh = plsc.VectorSubcoreMesh(core_axis_name="c", subcore_axis_name="s")
    fn = pl.kernel(
        _sc_gather_body,
        out_type=[jax.ShapeDtypeStruct((4, NP_PAD * 27), _f32),
                  jax.ShapeDtypeStruct((NP_PAD * 27,), _i32)],
        mesh=mesh,
        scratch_types=[pltpu.VMEM((P_CH,), _i32),
                       pltpu.VMEM((P_CH,), _i32),
                       pltpu.VMEM((P_CH,), _i32),
                       pltpu.VMEM((LPC,), _i32),
                       pltpu.VMEM((LPC,), _i32),
                       pltpu.VMEM((LPC,), _f32),
                       pltpu.SemaphoreType.DMA],
        compiler_params=pltpu.CompilerParams(use_tc_tiling_on_sc=False),
    )
    return fn(cx, cy, cz, tbl, bpad, x0, x1, x2, x3)


def _tc_contract_body(gx_ref, bj_ref, w1_ref, w2_ref, w3_ref, nlive_ref,
                      out_ref, t_ref):
    i = pl.program_id(0)

    @pl.when(i == 0)
    def _():
        t_ref[...] = jnp.zeros_like(t_ref)

    bj = bj_ref[...]
    kio = lax.broadcasted_iota(_i32, (32, BD), 0)
    pieces = []
    for j in range(27):
        row = jnp.broadcast_to(bj[j:j + 1, :], (32, BD))
        pieces.append((row >> kio) & 1)
    a = jnp.concatenate(pieces, axis=0).astype(_f32)
    t_ref[...] += lax.dot_general(a, gx_ref[...],
                                  (((1,), (1,)), ((), ())),
                                  preferred_element_type=_f32)

    @pl.when(i == N_BLK - 1)
    def _():
        t = t_ref[...]
        sm = lax.dot_general(t, w1_ref[...],
                             (((1,), (0,)), ((), ())),
                             preferred_element_type=_f32)
        u = jnp.zeros((32, 32), _f32)
        for j in range(27):
            u = u + lax.dot_general(sm[j * 32:(j + 1) * 32, :],
                                    w2_ref[j * 32:(j + 1) * 32, :],
                                    (((1,), (0,)), ((), ())),
                                    preferred_element_type=_f32)
        p3 = u[:, :, None] * w3_ref[...]
        res = jnp.sum(jnp.sum(p3, axis=0), axis=0)
        nl = nlive_ref[0, 0].astype(_f32)
        out_ref[...] = (res / nl)[None, :]


def _tc_contract(gxf, bj, w1r, w2r, w3p, nlive):
    return pl.pallas_call(
        _tc_contract_body,
        grid=(N_BLK,),
        in_specs=[pl.BlockSpec((108, BD), lambda i: (0, i)),
                  pl.BlockSpec((27, BD), lambda i: (0, i)),
                  pl.BlockSpec((108, 32), lambda i: (0, 0)),
                  pl.BlockSpec((864, 32), lambda i: (0, 0)),
                  pl.BlockSpec((32, 32, 32), lambda i: (0, 0, 0)),
                  pl.BlockSpec(memory_space=pltpu.SMEM)],
        out_specs=pl.BlockSpec((1, 32), lambda i: (0, 0)),
        out_shape=jax.ShapeDtypeStruct((1, 32), _f32),
        scratch_shapes=[pltpu.VMEM((864, 108), _f32)],
    )(gxf, bj, w1r, w2r, w3p, nlive)


@jax.jit
def kernel(x, coords, W1, W2, W3):
    coords = coords.astype(_i32)
    n = x.shape[0]

    pad = jnp.full((NP_PAD - n, 3), -4, _i32)
    cp = jnp.concatenate([coords, pad], axis=0)
    cx = cp[:, 0] + 0
    cy = cp[:, 1] + 0
    cz = cp[:, 2] + 0
    xcat = jnp.concatenate([x, jnp.zeros((8, 4), _f32)], axis=0)
    x0, x1, x2, x3 = (xcat[:, d] + 0 for d in range(4))

    tbl, occ = _sc_tables(cx, cy, cz)
    bmask, nlive = _tc_bmask(occ[:C3].reshape(64, 64, 64))
    bpad = jnp.concatenate([bmask.reshape(-1), jnp.zeros((8,), _i32)])

    gxp, bjf = _sc_gather(cx, cy, cz, tbl, bpad, x0, x1, x2, x3)
    tch = NW * N_CH
    gxf = gxp.reshape(4, tch, 27, P_CH).transpose(0, 2, 1, 3).reshape(108, NP_PAD)
    bj = bjf.reshape(tch, 27, P_CH).transpose(1, 0, 2).reshape(27, NP_PAD)

    w1r = W1.transpose(1, 0, 2).reshape(108, 32)
    w2r = W2.reshape(864, 32)
    w3p = jnp.concatenate([W3, jnp.zeros((5, 32, 32), _f32)], axis=0)
    out = _tc_contract(gxf, bj, w1r, w2r, w3p, nlive)
    return out[0]
